# speedup vs baseline: 2.2024x; 2.2024x over previous
import functools

import jax
import jax.numpy as jnp
from jax import lax
from jax.experimental import pallas as pl
from jax.experimental.pallas import tpu as pltpu
from jax.experimental.pallas import tpu_sc as plsc

_NUM_CORES = 2
_NUM_SUBCORES = 16
_LANES = 16
_NW = _NUM_CORES * _NUM_SUBCORES

_VOCAB = 100000
_MAXLEN = 4096
_D = 128
_BATCH = 4
_SEQ = 4096
_N = _BATCH * _SEQ
_PER_W = _N // _NW
_CHUNK = 128
_NCHUNK = _PER_W // _CHUNK


def _embed_body(tok_tab, pos_tab, tid, pid, out,
                tidx_v, pidx_v, tok_rows, pos_rows, sem_t, sem_p):
    c = lax.axis_index("c")
    s = lax.axis_index("s")
    wid = s * _NUM_CORES + c
    pltpu.sync_copy(tid.at[wid], tidx_v)
    pltpu.sync_copy(pid.at[wid], pidx_v)
    for j in range(_NCHUNK):
        ct = pltpu.async_copy(tok_tab.at[tidx_v.at[j]], tok_rows, sem_t)
        cp = pltpu.async_copy(pos_tab.at[pidx_v.at[j]], pos_rows, sem_p)
        ct.wait()
        cp.wait()

        def add_row(i, _):
            for q in range(_D // _LANES):
                sl = pl.ds(q * _LANES, _LANES)
                tok_rows[i, sl] = tok_rows[i, sl] + pos_rows[i, sl]
            return 0

        lax.fori_loop(0, _CHUNK, add_row, 0)
        pltpu.sync_copy(tok_rows,
                        out.at[pl.ds(wid * _PER_W + j * _CHUNK, _CHUNK)])


def _embed(tok_table, pos_table, tid, pid):
    mesh = plsc.VectorSubcoreMesh(core_axis_name="c", subcore_axis_name="s")
    return pl.kernel(
        _embed_body,
        out_type=jax.ShapeDtypeStruct((_N, _D), jnp.float32),
        mesh=mesh,
        scratch_types=[
            pltpu.VMEM((_NCHUNK, _CHUNK), jnp.int32),
            pltpu.VMEM((_NCHUNK, _CHUNK), jnp.int32),
            pltpu.VMEM((_CHUNK, _D), jnp.float32),
            pltpu.VMEM((_CHUNK, _D), jnp.float32),
            pltpu.SemaphoreType.DMA,
            pltpu.SemaphoreType.DMA,
        ],
    )(tok_table, pos_table, tid, pid)


def kernel(token_ids, position_ids, tok_table, pos_table):
    tid = token_ids.reshape(_NW, _NCHUNK, _CHUNK)
    pid = position_ids.reshape(_NW, _NCHUNK, _CHUNK)
    out = _embed(tok_table, pos_table, tid, pid)
    return out.reshape(_BATCH, _SEQ, _D)
